# baseline (speedup 1.0000x reference)
import jax
import jax.numpy as jnp
from jax import lax
from jax.experimental import pallas as pl
from jax.experimental.pallas import tpu as pltpu
from jax.experimental.pallas import tpu_sc as plsc

_NW = 32
_C = 2016
_CHUNKS = 31
_W = _C * _CHUNKS
_N = 2_000_000
_TAIL = _N - _NW * _W
_TBL = 64


def _sc_body(x_hbm, z_hbm, t_hbm, out_hbm, tbl_v, idx_v, xv):
    wid = lax.axis_index("s") * 2 + lax.axis_index("c")
    pltpu.sync_copy(t_hbm, tbl_v)

    def do_span(base, n):
        pltpu.sync_copy(z_hbm.at[pl.ds(base, n)], idx_v.at[pl.ds(0, n)])
        pltpu.sync_copy(x_hbm.at[pl.ds(base, n)], xv.at[pl.ds(0, n)])

        def body(v, carry):
            s = v * 16
            idx = idx_v[pl.ds(s, 16)]
            sh = plsc.load_gather(tbl_v, [idx])
            xv[pl.ds(s, 16)] = xv[pl.ds(s, 16)] + sh
            return carry

        lax.fori_loop(0, n // 16, body, 0)
        pltpu.sync_copy(xv.at[pl.ds(0, n)], out_hbm.at[pl.ds(base, n)])

    for c in range(_CHUNKS):
        do_span(wid * _W + c * _C, _C)

    @pl.when(wid == 0)
    def _tail():
        do_span(_NW * _W, _TAIL)


def kernel(inputs, z, shift_table):
    n = inputs.shape[0]
    x = inputs.reshape(n)
    zi = z.astype(jnp.int32)
    tbl = jnp.zeros((_TBL,), jnp.float32)
    tbl = tbl.at[: shift_table.shape[0]].set(shift_table.reshape(-1))
    mesh = plsc.VectorSubcoreMesh(core_axis_name="c", subcore_axis_name="s")
    out = pl.kernel(
        _sc_body,
        out_type=jax.ShapeDtypeStruct((n,), jnp.float32),
        mesh=mesh,
        compiler_params=pltpu.CompilerParams(needs_layout_passes=False),
        scratch_types=[
            pltpu.VMEM((_TBL,), jnp.float32),
            pltpu.VMEM((_C,), jnp.int32),
            pltpu.VMEM((_C,), jnp.float32),
        ],
    )(x, zi, tbl)
    return out.reshape(n, 1)


# speedup vs baseline: 1.3744x; 1.3744x over previous
import jax
import jax.numpy as jnp
from jax import lax
from jax.experimental import pallas as pl
from jax.experimental.pallas import tpu as pltpu
from jax.experimental.pallas import tpu_sc as plsc

_NW = 32
_C = 4464
_CHUNKS = 14
_W = _C * _CHUNKS
_N = 2_000_000
_TAIL = _N - _NW * _W
_TBL = 64
_VECS = _C // 16
_UNROLL = 9


def _sc_body(x_hbm, z_hbm, t_hbm, out_hbm, tbl_v,
             zb0, zb1, xb0, xb1, ob0, ob1,
             sz0, sz1, sx0, sx1, so0, so1):
    wid = lax.axis_index("s") * 2 + lax.axis_index("c")
    pltpu.sync_copy(t_hbm, tbl_v)
    zb, xb, ob = (zb0, zb1), (xb0, xb1), (ob0, ob1)
    sz, sx, so = (sz0, sz1), (sx0, sx1), (so0, so1)
    base0 = wid * _W

    def start_in(c, b):
        off = base0 + c * _C
        dz = pltpu.async_copy(z_hbm.at[pl.ds(off, _C)], zb[b], sz[b])
        dx = pltpu.async_copy(x_hbm.at[pl.ds(off, _C)], xb[b], sx[b])
        return dz, dx

    in_d = {0: start_in(0, 0)}
    out_d = {}
    for c in range(_CHUNKS):
        cur = c & 1
        if c + 1 < _CHUNKS:
            in_d[c + 1] = start_in(c + 1, cur ^ 1)
        dz, dx = in_d.pop(c)
        dz.wait()
        dx.wait()
        if c >= 2:
            out_d.pop(c - 2).wait()
        zv, xv, ov = zb[cur], xb[cur], ob[cur]

        @plsc.parallel_loop(0, _VECS, 1, unroll=_UNROLL)
        def _compute(v):
            s = v * 16
            idx = zv[pl.ds(s, 16)]
            sh = plsc.load_gather(tbl_v, [idx])
            ov[pl.ds(s, 16)] = xv[pl.ds(s, 16)] + sh

        out_d[c] = pltpu.async_copy(
            ob[cur], out_hbm.at[pl.ds(base0 + c * _C, _C)], so[cur])

    for c in sorted(out_d):
        out_d[c].wait()

    @pl.when(wid == 0)
    def _tail():
        toff = _NW * _W
        pltpu.sync_copy(z_hbm.at[pl.ds(toff, _TAIL)], zb0.at[pl.ds(0, _TAIL)])
        pltpu.sync_copy(x_hbm.at[pl.ds(toff, _TAIL)], xb0.at[pl.ds(0, _TAIL)])

        def body(v, carry):
            s = v * 16
            idx = zb0[pl.ds(s, 16)]
            sh = plsc.load_gather(tbl_v, [idx])
            ob0[pl.ds(s, 16)] = xb0[pl.ds(s, 16)] + sh
            return carry

        lax.fori_loop(0, _TAIL // 16, body, 0)
        pltpu.sync_copy(ob0.at[pl.ds(0, _TAIL)], out_hbm.at[pl.ds(toff, _TAIL)])


def kernel(inputs, z, shift_table):
    n = inputs.shape[0]
    x = inputs.reshape(n)
    zi = z.astype(jnp.int32)
    tbl = jnp.zeros((_TBL,), jnp.float32)
    tbl = tbl.at[: shift_table.shape[0]].set(shift_table.reshape(-1))
    mesh = plsc.VectorSubcoreMesh(core_axis_name="c", subcore_axis_name="s")
    out = pl.kernel(
        _sc_body,
        out_type=jax.ShapeDtypeStruct((n,), jnp.float32),
        mesh=mesh,
        compiler_params=pltpu.CompilerParams(needs_layout_passes=False),
        scratch_types=[
            pltpu.VMEM((_TBL,), jnp.float32),
            pltpu.VMEM((_C,), jnp.int32),
            pltpu.VMEM((_C,), jnp.int32),
            pltpu.VMEM((_C,), jnp.float32),
            pltpu.VMEM((_C,), jnp.float32),
            pltpu.VMEM((_C,), jnp.float32),
            pltpu.VMEM((_C,), jnp.float32),
            pltpu.SemaphoreType.DMA,
            pltpu.SemaphoreType.DMA,
            pltpu.SemaphoreType.DMA,
            pltpu.SemaphoreType.DMA,
            pltpu.SemaphoreType.DMA,
            pltpu.SemaphoreType.DMA,
        ],
    )(x, zi, tbl)
    return out.reshape(n, 1)
